# baseline (speedup 1.0000x reference)
import functools

import jax
import jax.numpy as jnp
from jax import lax
from jax.experimental import pallas as pl
from jax.experimental.pallas import tpu as pltpu
from jax.experimental.pallas import tpu_sc as plsc

_NC = 2
_NS = 16
_NW = _NC * _NS
_CH = 128


def _node_proj(nfeat, w1_src, w1_dst):
    n, d = nfeat.shape
    h = w1_src.shape[1]

    def body(n_ref, ws_ref, wd_ref, t_ref):
        x = n_ref[...]
        t_ref[0:n, :] = jnp.dot(
            x, ws_ref[...], preferred_element_type=jnp.float32)
        t_ref[n:2 * n, :] = jnp.dot(
            x, wd_ref[...], preferred_element_type=jnp.float32)

    return pl.pallas_call(
        body,
        out_shape=jax.ShapeDtypeStruct((2 * n, h), jnp.float32),
    )(nfeat, w1_src, w1_dst)


def _sc_gather(table, idx, out_rows, cpw0, cpw1):
    d = table.shape[1]
    cpw_max = max(cpw0, cpw1)
    mesh = plsc.VectorSubcoreMesh(core_axis_name="c", subcore_axis_name="s")

    @functools.partial(
        pl.kernel,
        mesh=mesh,
        out_type=jax.ShapeDtypeStruct((out_rows, d), jnp.float32),
        scratch_types=[
            pltpu.VMEM((cpw_max * _CH,), jnp.int32),
            pltpu.VMEM((_CH, d), jnp.float32),
            pltpu.VMEM((_CH, d), jnp.float32),
            pltpu.SemaphoreType.DMA,
            pltpu.SemaphoreType.DMA,
            pltpu.SemaphoreType.DMA,
        ],
    )
    def gather_k(t_hbm, idx_hbm, out_hbm, idx_v, r0, r1, gsem, w0, w1):
        c = lax.axis_index("c")
        s = lax.axis_index("s")
        my_cpw = jnp.where(c == 0, cpw0, cpw1)
        base_chunk = jnp.where(c == 0, s * cpw0, _NS * cpw0 + s * cpw1)
        base = base_chunk * _CH
        pltpu.sync_copy(idx_hbm.at[pl.ds(base, cpw_max * _CH)], idx_v)

        @pl.loop(0, cpw_max, step=2)
        def _(j):
            for b, r, ws in ((0, r0, w0), (1, r1, w1)):
                jj = j + b

                @pl.when(jj < my_cpw)
                def _():
                    @pl.when(jj >= 2)
                    def _():
                        pltpu.make_async_copy(
                            r, out_hbm.at[pl.ds(base + (jj - 2) * _CH, _CH)],
                            ws).wait()

                    pltpu.async_copy(
                        t_hbm.at[idx_v.at[pl.ds(jj * _CH, _CH)]], r, gsem
                    ).wait()
                    pltpu.async_copy(
                        r, out_hbm.at[pl.ds(base + jj * _CH, _CH)], ws)

        pltpu.make_async_copy(
            r0, out_hbm.at[pl.ds(base + (my_cpw - 2) * _CH, _CH)], w0).wait()
        pltpu.make_async_copy(
            r1, out_hbm.at[pl.ds(base + (my_cpw - 1) * _CH, _CH)], w1).wait()

    return gather_k(table, idx)


def _mlp_core(x_ref, rs_ref, rd_ref, we_ref, w2_ref, b1_ref, b2_ref,
              g_ref, bb_ref, o_ref):
    x = x_ref[...]
    h = jnp.dot(x, we_ref[...], preferred_element_type=jnp.float32)
    h = h + rs_ref[...] + rd_ref[...] + b1_ref[...]
    h = h * jax.nn.sigmoid(h)
    h2 = jnp.dot(h, w2_ref[...], preferred_element_type=jnp.float32)
    h2 = h2 + b2_ref[...]
    mu = jnp.mean(h2, axis=-1, keepdims=True)
    var = jnp.mean((h2 - mu) * (h2 - mu), axis=-1, keepdims=True)
    o_ref[...] = (h2 - mu) * lax.rsqrt(var + 1e-5) * g_ref[...] + bb_ref[...] + x


def _edge_mlp_seg(efeat, rows, w1_edge, w2, b1, b2, ln_g, ln_b,
                  block, seg, nseg, buf):
    e, d = efeat.shape
    e_seg = e // nseg
    nblk = e_seg // block
    base_blk = seg * nblk
    hid = w1_edge.shape[1]
    out_dim = w2.shape[1]
    full = lambda *s: pl.BlockSpec(s, lambda i: tuple(0 for _ in s))
    in_specs = [
        pl.BlockSpec((block, d), lambda i: (base_blk + i, 0)),
        pl.BlockSpec((block, hid), lambda i: (i, 0)),
        pl.BlockSpec((block, hid), lambda i: (i + nblk, 0)),
        full(d, hid),
        full(hid, out_dim),
        full(1, hid),
        full(1, out_dim),
        full(1, out_dim),
        full(1, out_dim),
    ]
    operands = (efeat, rows, rows, w1_edge, w2, b1, b2, ln_g, ln_b)
    kwargs = {}
    body = _mlp_core
    if buf is not None:
        def body(x, rs, rd, we, w2r, b1r, b2r, gr, bbr, _buf, o):
            _mlp_core(x, rs, rd, we, w2r, b1r, b2r, gr, bbr, o)
        in_specs = in_specs + [pl.BlockSpec(memory_space=pl.ANY)]
        operands = operands + (buf,)
        kwargs = dict(input_output_aliases={9: 0})
    return pl.pallas_call(
        body,
        grid=(nblk,),
        in_specs=in_specs,
        out_specs=pl.BlockSpec((block, out_dim), lambda i: (base_blk + i, 0)),
        out_shape=jax.ShapeDtypeStruct((e, out_dim), jnp.float32),
        compiler_params=pltpu.CompilerParams(
            dimension_semantics=("arbitrary",),
        ),
        **kwargs,
    )(*operands)


def kernel(efeat, nfeat, edge_index, W1, b1, W2, b2, ln_g, ln_b):
    e, d_edge = efeat.shape
    n, d_node = nfeat.shape
    src = edge_index[0]
    dst = edge_index[1]

    table = _node_proj(nfeat, W1[d_edge:d_edge + d_node], W1[d_edge + d_node:])

    nseg = 4
    e_seg = e // nseg
    quantum = 2 * _NW * _CH
    total = ((2 * e_seg + quantum - 1) // quantum) * quantum
    seg_chunks = total // _CH
    cpw0 = (int(seg_chunks * 0.72) // (2 * _NS)) * 2
    cpw1 = seg_chunks // _NS - cpw0
    pad = total - 2 * e_seg + (cpw0 - cpw1) * _CH
    zpad = jnp.zeros((pad,), dtype=jnp.int32)
    b1r, b2r = b1.reshape(1, -1), b2.reshape(1, -1)
    gr, br = ln_g.reshape(1, -1), ln_b.reshape(1, -1)
    w1e = W1[:d_edge]

    buf = None
    for k in range(nseg):
        sk = lax.slice(src, (k * e_seg,), ((k + 1) * e_seg,))
        dk = lax.slice(dst, (k * e_seg,), ((k + 1) * e_seg,))
        idx_k = jnp.concatenate([sk, dk + n, zpad])
        rows_k = _sc_gather(table, idx_k, total, cpw0, cpw1)
        buf = _edge_mlp_seg(efeat, rows_k, w1e, W2, b1r, b2r, gr, br,
                            block=1000, seg=k, nseg=nseg, buf=buf)
    return (buf, nfeat)


# speedup vs baseline: 1.0571x; 1.0571x over previous
import functools

import jax
import jax.numpy as jnp
from jax import lax
from jax.experimental import pallas as pl
from jax.experimental.pallas import tpu as pltpu
from jax.experimental.pallas import tpu_sc as plsc

_NC = 2
_NS = 16
_NW = _NC * _NS
_CH = 128


def _node_proj(nfeat, w1_src, w1_dst):
    n, d = nfeat.shape
    h = w1_src.shape[1]

    def body(n_ref, ws_ref, wd_ref, t_ref):
        x = n_ref[...]
        t_ref[0:n, :] = jnp.dot(
            x, ws_ref[...], preferred_element_type=jnp.float32)
        t_ref[n:2 * n, :] = jnp.dot(
            x, wd_ref[...], preferred_element_type=jnp.float32)

    return pl.pallas_call(
        body,
        out_shape=jax.ShapeDtypeStruct((2 * n, h), jnp.float32),
    )(nfeat, w1_src, w1_dst)


def _sc_gather(table, idx, out_rows, cpw0, cpw1):
    d = table.shape[1]
    cpw_max = max(cpw0, cpw1)
    mesh = plsc.VectorSubcoreMesh(core_axis_name="c", subcore_axis_name="s")
    nbuf = 4

    @functools.partial(
        pl.kernel,
        mesh=mesh,
        out_type=jax.ShapeDtypeStruct((out_rows, d), jnp.float32),
        scratch_types=[pltpu.VMEM((cpw_max * _CH,), jnp.int32)]
        + [pltpu.VMEM((_CH, d), jnp.float32)] * nbuf
        + [pltpu.SemaphoreType.DMA] * (2 * nbuf),
    )
    def gather_k(t_hbm, idx_hbm, out_hbm, idx_v, *bufs_and_sems):
        rbufs = bufs_and_sems[:nbuf]
        gsems = bufs_and_sems[nbuf:2 * nbuf]
        wsems = bufs_and_sems[2 * nbuf:]
        c = lax.axis_index("c")
        s = lax.axis_index("s")
        my_cpw = jnp.where(c == 0, cpw0, cpw1)
        base_chunk = jnp.where(c == 0, s * cpw0, _NS * cpw0 + s * cpw1)
        base = base_chunk * _CH
        pltpu.sync_copy(idx_hbm.at[pl.ds(base, cpw_max * _CH)], idx_v)

        def gat(jj, b):
            return pltpu.make_async_copy(
                t_hbm.at[idx_v.at[pl.ds(jj * _CH, _CH)]], rbufs[b], gsems[b])

        def wrb(jj, b):
            return pltpu.make_async_copy(
                rbufs[b], out_hbm.at[pl.ds(base + jj * _CH, _CH)], wsems[b])

        gat(0, 0).start()
        gat(1, 1).start()

        @pl.loop(0, cpw_max, step=nbuf)
        def _(j):
            for b in range(nbuf):
                jj = j + b

                @pl.when(jj < my_cpw)
                def _():
                    gat(jj, b).wait()
                    wrb(jj, b).start()
                    jn = jj + 2
                    bn = (b + 2) % nbuf

                    @pl.when(jn < my_cpw)
                    def _():
                        @pl.when(jj >= 2)
                        def _():
                            wrb(jj - 2, bn).wait()

                        gat(jn, bn).start()

        for b in range(nbuf):
            wrb(my_cpw - nbuf + b, b).wait()

    return gather_k(table, idx)


def _mlp_core(x_ref, rs_ref, rd_ref, we_ref, w2_ref, b1_ref, b2_ref,
              g_ref, bb_ref, o_ref):
    x = x_ref[...]
    h = jnp.dot(x, we_ref[...], preferred_element_type=jnp.float32)
    h = h + rs_ref[...] + rd_ref[...] + b1_ref[...]
    h = h * jax.nn.sigmoid(h)
    h2 = jnp.dot(h, w2_ref[...], preferred_element_type=jnp.float32)
    h2 = h2 + b2_ref[...]
    mu = jnp.mean(h2, axis=-1, keepdims=True)
    var = jnp.mean((h2 - mu) * (h2 - mu), axis=-1, keepdims=True)
    o_ref[...] = (h2 - mu) * lax.rsqrt(var + 1e-5) * g_ref[...] + bb_ref[...] + x


def _edge_mlp_seg(efeat, rows, w1_edge, w2, b1, b2, ln_g, ln_b,
                  block, seg, nseg, buf):
    e, d = efeat.shape
    e_seg = e // nseg
    nblk = e_seg // block
    base_blk = seg * nblk
    hid = w1_edge.shape[1]
    out_dim = w2.shape[1]
    full = lambda *s: pl.BlockSpec(s, lambda i: tuple(0 for _ in s))
    in_specs = [
        pl.BlockSpec((block, d), lambda i: (base_blk + i, 0)),
        pl.BlockSpec((block, hid), lambda i: (i, 0)),
        pl.BlockSpec((block, hid), lambda i: (i + nblk, 0)),
        full(d, hid),
        full(hid, out_dim),
        full(1, hid),
        full(1, out_dim),
        full(1, out_dim),
        full(1, out_dim),
    ]
    operands = (efeat, rows, rows, w1_edge, w2, b1, b2, ln_g, ln_b)
    kwargs = {}
    body = _mlp_core
    if buf is not None:
        def body(x, rs, rd, we, w2r, b1r, b2r, gr, bbr, _buf, o):
            _mlp_core(x, rs, rd, we, w2r, b1r, b2r, gr, bbr, o)
        in_specs = in_specs + [pl.BlockSpec(memory_space=pl.ANY)]
        operands = operands + (buf,)
        kwargs = dict(input_output_aliases={9: 0})
    return pl.pallas_call(
        body,
        grid=(nblk,),
        in_specs=in_specs,
        out_specs=pl.BlockSpec((block, out_dim), lambda i: (base_blk + i, 0)),
        out_shape=jax.ShapeDtypeStruct((e, out_dim), jnp.float32),
        compiler_params=pltpu.CompilerParams(
            dimension_semantics=("arbitrary",),
        ),
        **kwargs,
    )(*operands)


def kernel(efeat, nfeat, edge_index, W1, b1, W2, b2, ln_g, ln_b):
    e, d_edge = efeat.shape
    n, d_node = nfeat.shape
    src = edge_index[0]
    dst = edge_index[1]

    table = _node_proj(nfeat, W1[d_edge:d_edge + d_node], W1[d_edge + d_node:])

    nseg = 4
    e_seg = e // nseg
    quantum = 2 * _NW * _CH
    total = ((2 * e_seg + quantum - 1) // quantum) * quantum
    seg_chunks = total // _CH
    cpw0 = (int(seg_chunks * 0.8) // (4 * _NS)) * 4
    cpw1 = seg_chunks // _NS - cpw0
    pad = total - 2 * e_seg + (cpw0 - cpw1) * _CH
    zpad = jnp.zeros((pad,), dtype=jnp.int32)
    b1r, b2r = b1.reshape(1, -1), b2.reshape(1, -1)
    gr, br = ln_g.reshape(1, -1), ln_b.reshape(1, -1)
    w1e = W1[:d_edge]

    buf = None
    for k in range(nseg):
        sk = lax.slice(src, (k * e_seg,), ((k + 1) * e_seg,))
        dk = lax.slice(dst, (k * e_seg,), ((k + 1) * e_seg,))
        idx_k = jnp.concatenate([sk, dk + n, zpad])
        rows_k = _sc_gather(table, idx_k, total, cpw0, cpw1)
        buf = _edge_mlp_seg(efeat, rows_k, w1e, W2, b1r, b2r, gr, br,
                            block=1000, seg=k, nseg=nseg, buf=buf)
    return (buf, nfeat)


# speedup vs baseline: 1.0751x; 1.0170x over previous
import functools

import jax
import jax.numpy as jnp
from jax import lax
from jax.experimental import pallas as pl
from jax.experimental.pallas import tpu as pltpu
from jax.experimental.pallas import tpu_sc as plsc

_NC = 2
_NS = 16
_NW = _NC * _NS
_CH = 128


def _node_proj(nfeat, w1_src, w1_dst):
    n, d = nfeat.shape
    h = w1_src.shape[1]

    def body(n_ref, ws_ref, wd_ref, t_ref):
        x = n_ref[...]
        t_ref[0:n, :] = jnp.dot(
            x, ws_ref[...], preferred_element_type=jnp.float32)
        t_ref[n:2 * n, :] = jnp.dot(
            x, wd_ref[...], preferred_element_type=jnp.float32)

    return pl.pallas_call(
        body,
        out_shape=jax.ShapeDtypeStruct((2 * n, h), jnp.float32),
    )(nfeat, w1_src, w1_dst)


def _sc_gather(table, idx, out_rows, cpw0, cpw1):
    d = table.shape[1]
    cpw_max = max(cpw0, cpw1)
    mesh = plsc.VectorSubcoreMesh(core_axis_name="c", subcore_axis_name="s")
    nbuf = 4

    @functools.partial(
        pl.kernel,
        mesh=mesh,
        out_type=jax.ShapeDtypeStruct((out_rows, d), jnp.float32),
        scratch_types=[pltpu.VMEM((cpw_max * _CH,), jnp.int32)]
        + [pltpu.VMEM((_CH, d), jnp.float32)] * nbuf
        + [pltpu.SemaphoreType.DMA] * (2 * nbuf),
    )
    def gather_k(t_hbm, idx_hbm, out_hbm, idx_v, *bufs_and_sems):
        rbufs = bufs_and_sems[:nbuf]
        gsems = bufs_and_sems[nbuf:2 * nbuf]
        wsems = bufs_and_sems[2 * nbuf:]
        c = lax.axis_index("c")
        s = lax.axis_index("s")
        my_cpw = jnp.where(c == 0, cpw0, cpw1)
        base_chunk = jnp.where(c == 0, s * cpw0, _NS * cpw0 + s * cpw1)
        base = base_chunk * _CH
        pltpu.sync_copy(idx_hbm.at[pl.ds(base, cpw_max * _CH)], idx_v)

        def gat(jj, b):
            return pltpu.make_async_copy(
                t_hbm.at[idx_v.at[pl.ds(jj * _CH, _CH)]], rbufs[b], gsems[b])

        def wrb(jj, b):
            return pltpu.make_async_copy(
                rbufs[b], out_hbm.at[pl.ds(base + jj * _CH, _CH)], wsems[b])

        gat(0, 0).start()
        gat(1, 1).start()

        @pl.loop(0, cpw_max, step=nbuf)
        def _(j):
            for b in range(nbuf):
                jj = j + b

                @pl.when(jj < my_cpw)
                def _():
                    gat(jj, b).wait()
                    wrb(jj, b).start()
                    jn = jj + 2
                    bn = (b + 2) % nbuf

                    @pl.when(jn < my_cpw)
                    def _():
                        @pl.when(jj >= 2)
                        def _():
                            wrb(jj - 2, bn).wait()

                        gat(jn, bn).start()

        for b in range(nbuf):
            wrb(my_cpw - nbuf + b, b).wait()

    return gather_k(table, idx)


def _mlp_core(x_ref, rs_ref, rd_ref, we_ref, w2_ref, b1_ref, b2_ref,
              g_ref, bb_ref, o_ref):
    x = x_ref[...]
    h = jnp.dot(x, we_ref[...], preferred_element_type=jnp.float32)
    h = h + rs_ref[...] + rd_ref[...] + b1_ref[...]
    h = h * jax.nn.sigmoid(h)
    h2 = jnp.dot(h, w2_ref[...], preferred_element_type=jnp.float32)
    h2 = h2 + b2_ref[...]
    mu = jnp.mean(h2, axis=-1, keepdims=True)
    var = jnp.mean((h2 - mu) * (h2 - mu), axis=-1, keepdims=True)
    o_ref[...] = (h2 - mu) * lax.rsqrt(var + 1e-5) * g_ref[...] + bb_ref[...] + x


def _edge_mlp_seg(efeat, rows, w1_edge, w2, b1, b2, ln_g, ln_b,
                  block, seg, nseg, buf):
    e, d = efeat.shape
    e_seg = e // nseg
    nblk = e_seg // block
    base_blk = seg * nblk
    hid = w1_edge.shape[1]
    out_dim = w2.shape[1]
    full = lambda *s: pl.BlockSpec(s, lambda i: tuple(0 for _ in s))
    in_specs = [
        pl.BlockSpec((block, d), lambda i: (base_blk + i, 0)),
        pl.BlockSpec((block, hid), lambda i: (i, 0)),
        pl.BlockSpec((block, hid), lambda i: (i + nblk, 0)),
        full(d, hid),
        full(hid, out_dim),
        full(1, hid),
        full(1, out_dim),
        full(1, out_dim),
        full(1, out_dim),
    ]
    operands = (efeat, rows, rows, w1_edge, w2, b1, b2, ln_g, ln_b)
    kwargs = {}
    body = _mlp_core
    if buf is not None:
        def body(x, rs, rd, we, w2r, b1r, b2r, gr, bbr, _buf, o):
            _mlp_core(x, rs, rd, we, w2r, b1r, b2r, gr, bbr, o)
        in_specs = in_specs + [pl.BlockSpec(memory_space=pl.ANY)]
        operands = operands + (buf,)
        kwargs = dict(input_output_aliases={9: 0})
    return pl.pallas_call(
        body,
        grid=(nblk,),
        in_specs=in_specs,
        out_specs=pl.BlockSpec((block, out_dim), lambda i: (base_blk + i, 0)),
        out_shape=jax.ShapeDtypeStruct((e, out_dim), jnp.float32),
        compiler_params=pltpu.CompilerParams(
            dimension_semantics=("arbitrary",),
        ),
        **kwargs,
    )(*operands)


def kernel(efeat, nfeat, edge_index, W1, b1, W2, b2, ln_g, ln_b):
    e, d_edge = efeat.shape
    n, d_node = nfeat.shape
    src = edge_index[0]
    dst = edge_index[1]

    table = _node_proj(nfeat, W1[d_edge:d_edge + d_node], W1[d_edge + d_node:])

    nseg = 4
    e_seg = e // nseg
    quantum = 2 * _NW * _CH
    total = ((2 * e_seg + quantum - 1) // quantum) * quantum
    seg_chunks = total // _CH
    cpw0 = (int(seg_chunks * 0.9) // (4 * _NS)) * 4
    cpw1 = seg_chunks // _NS - cpw0
    pad = total - 2 * e_seg + (cpw0 - cpw1) * _CH
    zpad = jnp.zeros((pad,), dtype=jnp.int32)
    b1r, b2r = b1.reshape(1, -1), b2.reshape(1, -1)
    gr, br = ln_g.reshape(1, -1), ln_b.reshape(1, -1)
    w1e = W1[:d_edge]

    buf = None
    for k in range(nseg):
        sk = lax.slice(src, (k * e_seg,), ((k + 1) * e_seg,))
        dk = lax.slice(dst, (k * e_seg,), ((k + 1) * e_seg,))
        idx_k = jnp.concatenate([sk, dk + n, zpad])
        rows_k = _sc_gather(table, idx_k, total, cpw0, cpw1)
        buf = _edge_mlp_seg(efeat, rows_k, w1e, W2, b1r, b2r, gr, br,
                            block=1000, seg=k, nseg=nseg, buf=buf)
    return (buf, nfeat)
